# baseline (speedup 1.0000x reference)
import functools

import jax
import jax.numpy as jnp
from jax.experimental import pallas as pl

N_FEAT = 32
N_BINS = 64
N_PAR = 3 * N_BINS + 1
BLK = 512
RT = 512
CH = 128
NC = N_BINS * N_FEAT // CH


def _spline_subtile(x_ref, p_ref, x0, xf, y0, yf, y_ref, ld_ref, r,
                    li_masks):
    f32 = jnp.float32
    m_ge32, m_ge64, m_ge96, m_lt32, m_lt96 = li_masks
    r0 = r * RT
    x = x_ref[r0:r0 + RT, :]

    def pcols(lo, hi):
        return p_ref[r0:r0 + RT, lo:hi]

    def roll(v, n):
        return jnp.roll(v, n, axis=1)

    def lane_reduce(v):
        v2 = v[:, :64] + v[:, 64:]
        return v2[:, :N_FEAT] + v2[:, N_FEAT:]

    sacc = jnp.exp(pcols(0, CH))
    for kc in range(1, NC):
        sacc = sacc + jnp.exp(pcols(kc * CH, (kc + 1) * CH))
    sw32 = lane_reduce(sacc)
    tb32 = (x - x0) * sw32 / (xf - x0)
    tb = jnp.concatenate([tb32, tb32, tb32, tb32], axis=1)

    zero = jnp.zeros((RT, CH), f32)
    carry = zero
    prev_re = zero
    xk_acc = zero
    w_acc = zero
    sh_acc = zero
    yk_acc = zero
    h_acc = zero
    dk_acc = zero
    dk1_acc = zero
    off_h = N_BINS * N_FEAT
    off_s = 2 * N_BINS * N_FEAT
    for kc in range(NC):
        ew = jnp.exp(pcols(kc * CH, (kc + 1) * CH))
        a1 = ew + jnp.where(m_ge32, roll(ew, 32), 0.0)
        a2 = a1 + jnp.where(m_ge64, roll(a1, 64), 0.0)
        cumc = carry + a2
        cumprev = cumc - ew
        tot = jnp.where(m_ge96, a2, 0.0)
        tot = tot + roll(tot, 32)
        tot = tot + roll(tot, 64)
        carry = carry + tot
        lt = jnp.where(tb > cumc, 1.0, 0.0)
        if kc == NC - 1:
            lt = jnp.where(m_lt96, lt, 0.0)
        ltp = jnp.where(tb > cumprev, 1.0, 0.0)
        eq = ltp - lt
        xk_acc = xk_acc + cumprev * eq
        w_acc = w_acc + ew * eq
        eh = jnp.exp(pcols(off_h + kc * CH, off_h + (kc + 1) * CH))
        sh_acc = sh_acc + eh
        yk_acc = yk_acc + eh * lt
        h_acc = h_acc + eh * eq
        ps_c = pcols(off_s + kc * CH, off_s + (kc + 1) * CH)
        dk_acc = dk_acc + ps_c * eq
        re = roll(eq, 32)
        sh_eq = jnp.where(m_lt32, prev_re, re)
        dk1_acc = dk1_acc + ps_c * sh_eq
        prev_re = re
        prev_eq = eq

    xk_u = lane_reduce(xk_acc)
    w_u = lane_reduce(w_acc)
    sh32 = lane_reduce(sh_acc)
    yk_u = lane_reduce(yk_acc)
    h_u = lane_reduce(h_acc)
    dk_logit = lane_reduce(dk_acc)
    dk1_logit = lane_reduce(dk1_acc)

    eq63 = prev_eq[:, 96:]
    ps_last = pcols(N_PAR * N_FEAT - N_FEAT, N_PAR * N_FEAT)
    dk1_logit = dk1_logit + ps_last * eq63

    cxn = (xf - x0) / sw32
    cyn = (yf - y0) / sh32
    w = w_u * cxn
    xk = x0 + xk_u * cxn
    h = h_u * cyn
    yk = y0 + yk_u * cyn
    dk = jax.nn.softplus(dk_logit)
    dk1 = jax.nn.softplus(dk1_logit)

    s = h / w
    eps = (x - xk) / w
    e1me = eps * (1.0 - eps)
    e2 = eps * eps
    den = s + (dk1 + dk - 2.0 * s) * e1me
    y = yk + h * (s * e2 + dk * e1me) / den
    num_J = s * s * (dk1 * e2 + 2.0 * s * e1me + dk * (1.0 - eps) ** 2)
    y_ref[r0:r0 + RT, :] = y
    ld_ref[r0:r0 + RT, :] = jnp.sum(jnp.log(num_J / (den * den)), axis=1,
                                    keepdims=True)


def _spline_block_kernel(x_ref, p_ref, x0_ref, xf_ref, y0_ref, yf_ref,
                         y_ref, ld_ref):
    x0 = x0_ref[...]
    xf = xf_ref[...]
    y0 = y0_ref[...]
    yf = yf_ref[...]
    li = jax.lax.broadcasted_iota(jnp.int32, (1, CH), 1)
    li_masks = (li >= 32, li >= 64, li >= 96, li < 32, li < 96)
    for r in range(BLK // RT):
        _spline_subtile(x_ref, p_ref, x0, xf, y0, yf, y_ref, ld_ref, r,
                        li_masks)


N_SLICE = 4


@functools.partial(jax.jit, static_argnames=("interpret",))
def kernel(x, parameters, x0, xf, y0, yf, interpret=False):
    batch = x.shape[0]
    sb = batch // N_SLICE
    x0r = x0.reshape(1, -1)
    xfr = xf.reshape(1, -1)
    y0r = y0.reshape(1, -1)
    yfr = yf.reshape(1, -1)
    ys = []
    lds = []
    for s in range(N_SLICE):
        p2d = jax.lax.slice_in_dim(parameters, s * sb, (s + 1) * sb,
                                   axis=0).reshape(sb, N_PAR * N_FEAT)
        xs = jax.lax.slice_in_dim(x, s * sb, (s + 1) * sb, axis=0)
        y_s, ld_s = pl.pallas_call(
            _spline_block_kernel,
            grid=(sb // BLK,),
            in_specs=[
                pl.BlockSpec((BLK, N_FEAT), lambda i: (i, 0)),
                pl.BlockSpec((BLK, N_PAR * N_FEAT), lambda i: (i, 0)),
                pl.BlockSpec((1, N_FEAT), lambda i: (0, 0)),
                pl.BlockSpec((1, N_FEAT), lambda i: (0, 0)),
                pl.BlockSpec((1, N_FEAT), lambda i: (0, 0)),
                pl.BlockSpec((1, N_FEAT), lambda i: (0, 0)),
            ],
            out_specs=[
                pl.BlockSpec((BLK, N_FEAT), lambda i: (i, 0)),
                pl.BlockSpec((BLK, 1), lambda i: (i, 0)),
            ],
            out_shape=[
                jax.ShapeDtypeStruct((sb, N_FEAT), jnp.float32),
                jax.ShapeDtypeStruct((sb, 1), jnp.float32),
            ],
            interpret=interpret,
        )(xs, p2d, x0r, xfr, y0r, yfr)
        ys.append(y_s)
        lds.append(ld_s)
    y = jnp.concatenate(ys, axis=0)
    ld = jnp.concatenate(lds, axis=0)
    return y, ld.reshape(batch)


# speedup vs baseline: 1.4766x; 1.4766x over previous
import functools

import jax
import jax.numpy as jnp
from jax.experimental import pallas as pl

N_FEAT = 32
N_BINS = 64
N_PAR = 3 * N_BINS + 1
BLK = 512
RT = 512
CH = 128
NC = N_BINS * N_FEAT // CH


def _spline_subtile(x_ref, p_ref, x0, xf, y0, yf, y_ref, ld_ref, r,
                    li_masks):
    f32 = jnp.float32
    m_ge32, m_ge64, m_ge96, m_lt32, m_lt96 = li_masks
    r0 = r * RT
    x = x_ref[r0:r0 + RT, :]

    def pcols(lo, hi):
        return p_ref[r0:r0 + RT, lo:hi]

    def roll(v, n):
        return jnp.roll(v, n, axis=1)

    def lane_reduce(v):
        v2 = v[:, :64] + v[:, 64:]
        return v2[:, :N_FEAT] + v2[:, N_FEAT:]

    sacc = jnp.exp(pcols(0, CH))
    for kc in range(1, NC):
        sacc = sacc + jnp.exp(pcols(kc * CH, (kc + 1) * CH))
    sw32 = lane_reduce(sacc)
    tb32 = (x - x0) * sw32 / (xf - x0)
    tb = jnp.concatenate([tb32, tb32, tb32, tb32], axis=1)

    zero = jnp.zeros((RT, CH), f32)
    carry = zero
    prev_re = zero
    xk_acc = zero
    w_acc = zero
    sh_acc = zero
    yk_acc = zero
    h_acc = zero
    dk_acc = zero
    dk1_acc = zero
    off_h = N_BINS * N_FEAT
    off_s = 2 * N_BINS * N_FEAT
    for kc in range(NC):
        ew = jnp.exp(pcols(kc * CH, (kc + 1) * CH))
        a1 = ew + jnp.where(m_ge32, roll(ew, 32), 0.0)
        a2 = a1 + jnp.where(m_ge64, roll(a1, 64), 0.0)
        cumc = carry + a2
        cumprev = cumc - ew
        tot = jnp.where(m_ge96, a2, 0.0)
        tot = tot + roll(tot, 32)
        tot = tot + roll(tot, 64)
        carry = carry + tot
        lt = jnp.where(tb > cumc, 1.0, 0.0)
        if kc == NC - 1:
            lt = jnp.where(m_lt96, lt, 0.0)
        ltp = jnp.where(tb > cumprev, 1.0, 0.0)
        eq = ltp - lt
        xk_acc = xk_acc + cumprev * eq
        w_acc = w_acc + ew * eq
        eh = jnp.exp(pcols(off_h + kc * CH, off_h + (kc + 1) * CH))
        sh_acc = sh_acc + eh
        yk_acc = yk_acc + eh * lt
        h_acc = h_acc + eh * eq
        ps_c = pcols(off_s + kc * CH, off_s + (kc + 1) * CH)
        dk_acc = dk_acc + ps_c * eq
        re = roll(eq, 32)
        sh_eq = jnp.where(m_lt32, prev_re, re)
        dk1_acc = dk1_acc + ps_c * sh_eq
        prev_re = re
        prev_eq = eq

    xk_u = lane_reduce(xk_acc)
    w_u = lane_reduce(w_acc)
    sh32 = lane_reduce(sh_acc)
    yk_u = lane_reduce(yk_acc)
    h_u = lane_reduce(h_acc)
    dk_logit = lane_reduce(dk_acc)
    dk1_logit = lane_reduce(dk1_acc)

    eq63 = prev_eq[:, 96:]
    ps_last = pcols(N_PAR * N_FEAT - N_FEAT, N_PAR * N_FEAT)
    dk1_logit = dk1_logit + ps_last * eq63

    cxn = (xf - x0) / sw32
    cyn = (yf - y0) / sh32
    w = w_u * cxn
    xk = x0 + xk_u * cxn
    h = h_u * cyn
    yk = y0 + yk_u * cyn
    dk = jax.nn.softplus(dk_logit)
    dk1 = jax.nn.softplus(dk1_logit)

    s = h / w
    eps = (x - xk) / w
    e1me = eps * (1.0 - eps)
    e2 = eps * eps
    den = s + (dk1 + dk - 2.0 * s) * e1me
    y = yk + h * (s * e2 + dk * e1me) / den
    num_J = s * s * (dk1 * e2 + 2.0 * s * e1me + dk * (1.0 - eps) ** 2)
    y_ref[r0:r0 + RT, :] = y
    ld_ref[r0:r0 + RT, :] = jnp.sum(jnp.log(num_J / (den * den)), axis=1,
                                    keepdims=True)


def _spline_block_kernel(x_ref, p_ref, x0_ref, xf_ref, y0_ref, yf_ref,
                         y_ref, ld_ref):
    x0 = x0_ref[...]
    xf = xf_ref[...]
    y0 = y0_ref[...]
    yf = yf_ref[...]
    li = jax.lax.broadcasted_iota(jnp.int32, (1, CH), 1)
    li_masks = (li >= 32, li >= 64, li >= 96, li < 32, li < 96)
    for r in range(BLK // RT):
        _spline_subtile(x_ref, p_ref, x0, xf, y0, yf, y_ref, ld_ref, r,
                        li_masks)


@functools.partial(jax.jit, static_argnames=("interpret",))
def kernel(x, parameters, x0, xf, y0, yf, interpret=False):
    batch = x.shape[0]
    p2d = parameters.reshape(batch, N_PAR * N_FEAT)
    grid = (batch // BLK,)
    y, ld = pl.pallas_call(
        _spline_block_kernel,
        grid=grid,
        in_specs=[
            pl.BlockSpec((BLK, N_FEAT), lambda i: (i, 0)),
            pl.BlockSpec((BLK, N_PAR * N_FEAT), lambda i: (i, 0)),
            pl.BlockSpec((1, N_FEAT), lambda i: (0, 0)),
            pl.BlockSpec((1, N_FEAT), lambda i: (0, 0)),
            pl.BlockSpec((1, N_FEAT), lambda i: (0, 0)),
            pl.BlockSpec((1, N_FEAT), lambda i: (0, 0)),
        ],
        out_specs=[
            pl.BlockSpec((BLK, N_FEAT), lambda i: (i, 0)),
            pl.BlockSpec((BLK, 1), lambda i: (i, 0)),
        ],
        out_shape=[
            jax.ShapeDtypeStruct((batch, N_FEAT), jnp.float32),
            jax.ShapeDtypeStruct((batch, 1), jnp.float32),
        ],
        interpret=interpret,
    )(x, p2d, x0.reshape(1, -1), xf.reshape(1, -1),
      y0.reshape(1, -1), yf.reshape(1, -1))
    return y, ld.reshape(batch)
